# Initial kernel scaffold; baseline (speedup 1.0000x reference)
import jax
import jax.numpy as jnp
from jax.experimental import pallas as pl


def kernel(input_ids, token_type_ids, word_emb, pos_emb, type_emb, ln_gamma, ln_beta):
    raise NotImplementedError("write your pallas kernel here")


# speedup vs baseline: 1.5251x; 1.5251x over previous
import functools

import jax
import jax.numpy as jnp
from jax import lax
from jax.experimental import pallas as pl
from jax.experimental.pallas import tpu as pltpu
from jax.experimental.pallas import tpu_sc as plsc

EPS = 1e-12


def _sc_gather(word_emb, idx2d, n_ch, ch):
    V, D = word_emb.shape
    info = plsc.get_sparse_core_info()
    NC, NS = info.num_cores, info.num_subcores
    NW = NC * NS
    B = NW * n_ch * ch
    mesh = plsc.VectorSubcoreMesh(core_axis_name="c", subcore_axis_name="s")

    @functools.partial(
        pl.kernel,
        mesh=mesh,
        out_type=jax.ShapeDtypeStruct((B, D), jnp.float32),
        scratch_types=[
            pltpu.VMEM((n_ch, ch), jnp.int32),
            pltpu.VMEM((ch, D), jnp.float32),
            pltpu.VMEM((ch, D), jnp.float32),
            pltpu.SemaphoreType.DMA,
            pltpu.SemaphoreType.DMA,
            pltpu.SemaphoreType.DMA,
            pltpu.SemaphoreType.DMA,
        ],
    )
    def k(table_hbm, idx_hbm, out_hbm, idx_v, buf0, buf1, g0, g1, o0, o1):
        wid = lax.axis_index("s") * NC + lax.axis_index("c")
        base = wid * (n_ch * ch)
        pltpu.sync_copy(idx_hbm.at[pl.ds(wid * n_ch, n_ch)], idx_v)
        bufs = (buf0, buf1)
        gsems = (g0, g1)
        osems = (o0, o1)
        gcp = [None] * n_ch
        ocp = [None] * n_ch
        for c in range(n_ch):
            if c >= 2:
                ocp[c - 2].wait()
            gcp[c] = pltpu.async_copy(
                table_hbm.at[idx_v.at[c]], bufs[c % 2], gsems[c % 2])
            if c >= 1:
                gcp[c - 1].wait()
                ocp[c - 1] = pltpu.async_copy(
                    bufs[(c - 1) % 2],
                    out_hbm.at[pl.ds(base + (c - 1) * ch, ch)],
                    osems[(c - 1) % 2])
        last = n_ch - 1
        gcp[last].wait()
        ocp[last] = pltpu.async_copy(
            bufs[last % 2], out_hbm.at[pl.ds(base + last * ch, ch)],
            osems[last % 2])
        if n_ch >= 2:
            ocp[last - 1].wait()
        ocp[last].wait()

    return k(word_emb, idx2d)


def _tc_finish(gath, tt3, pos_emb, type_emb, gamma2, beta2, tb):
    T, D = gath.shape
    P = pos_emb.shape[0]
    grid = T // tb
    pos_blocks = P // tb

    def body(tt_ref, g_ref, pos_ref, type_ref, gam_ref, bet_ref, o_ref):
        tt = tt_ref[0, 0, :]
        typ = jnp.where((tt[:, None] == 0), type_ref[0, :][None, :],
                        type_ref[1, :][None, :])
        x = g_ref[...] + pos_ref[...] + typ
        mean = jnp.mean(x, axis=-1, keepdims=True)
        xc = x - mean
        var = jnp.mean(xc * xc, axis=-1, keepdims=True)
        inv = lax.rsqrt(var + EPS)
        o_ref[...] = xc * inv * gam_ref[0, :] + bet_ref[0, :]

    return pl.pallas_call(
        body,
        grid=(grid,),
        in_specs=[
            pl.BlockSpec((1, 1, tb), lambda i: (i, 0, 0)),
            pl.BlockSpec((tb, D), lambda i: (i, 0)),
            pl.BlockSpec((tb, D), lambda i: (i % pos_blocks, 0)),
            pl.BlockSpec((2, D), lambda i: (0, 0)),
            pl.BlockSpec((1, D), lambda i: (0, 0)),
            pl.BlockSpec((1, D), lambda i: (0, 0)),
        ],
        out_specs=pl.BlockSpec((tb, D), lambda i: (i, 0)),
        out_shape=jax.ShapeDtypeStruct((T, D), jnp.float32),
    )(tt3, gath, pos_emb, type_emb, gamma2, beta2)


def kernel(input_ids, token_type_ids, word_emb, pos_emb, type_emb, ln_gamma,
           ln_beta):
    B, S = input_ids.shape
    D = word_emb.shape[1]
    T = B * S

    n_ch, ch = 4, 64
    idx2d = input_ids.reshape(-1).astype(jnp.int32).reshape(-1, ch)
    gath = _sc_gather(word_emb, idx2d, n_ch, ch)

    tb = 256
    tt3 = token_type_ids.astype(jnp.int32).reshape(T // tb, 1, tb)
    out = _tc_finish(gath, tt3, pos_emb, type_emb,
                     ln_gamma.reshape(1, D), ln_beta.reshape(1, D), tb)
    return out.reshape(B, S, D)


# speedup vs baseline: 1.5558x; 1.0201x over previous
import functools

import jax
import jax.numpy as jnp
from jax import lax
from jax.experimental import pallas as pl
from jax.experimental.pallas import tpu as pltpu
from jax.experimental.pallas import tpu_sc as plsc

EPS = 1e-12


def _sc_gather(word_emb, idx2d, n_ch, ch):
    V, D = word_emb.shape
    info = plsc.get_sparse_core_info()
    NC, NS = info.num_cores, info.num_subcores
    NW = NC * NS
    B = NW * n_ch * ch
    mesh = plsc.VectorSubcoreMesh(core_axis_name="c", subcore_axis_name="s")

    @functools.partial(
        pl.kernel,
        mesh=mesh,
        out_type=jax.ShapeDtypeStruct((B, D), jnp.float32),
        scratch_types=[
            pltpu.VMEM((n_ch, ch), jnp.int32),
            pltpu.VMEM((ch, D), jnp.float32),
            pltpu.VMEM((ch, D), jnp.float32),
            pltpu.SemaphoreType.DMA,
            pltpu.SemaphoreType.DMA,
            pltpu.SemaphoreType.DMA,
            pltpu.SemaphoreType.DMA,
        ],
    )
    def k(table_hbm, idx_hbm, out_hbm, idx_v, buf0, buf1, g0, g1, o0, o1):
        wid = lax.axis_index("s") * NC + lax.axis_index("c")
        base = wid * (n_ch * ch)
        pltpu.sync_copy(idx_hbm.at[pl.ds(wid * n_ch, n_ch)], idx_v)
        bufs = (buf0, buf1)
        gsems = (g0, g1)
        osems = (o0, o1)
        gcp = [None] * n_ch
        ocp = [None] * n_ch
        for c in range(n_ch):
            if c >= 2:
                ocp[c - 2].wait()
            gcp[c] = pltpu.async_copy(
                table_hbm.at[idx_v.at[c]], bufs[c % 2], gsems[c % 2])
            if c >= 1:
                gcp[c - 1].wait()
                ocp[c - 1] = pltpu.async_copy(
                    bufs[(c - 1) % 2],
                    out_hbm.at[pl.ds(base + (c - 1) * ch, ch)],
                    osems[(c - 1) % 2])
        last = n_ch - 1
        gcp[last].wait()
        ocp[last] = pltpu.async_copy(
            bufs[last % 2], out_hbm.at[pl.ds(base + last * ch, ch)],
            osems[last % 2])
        if n_ch >= 2:
            ocp[last - 1].wait()
        ocp[last].wait()

    return k(word_emb, idx2d)


def _tc_finish(gath, tt3, pos_emb, type_emb, gamma2, beta2, tb):
    T, D = gath.shape
    P = pos_emb.shape[0]
    pos_blocks = P // tb
    batches = T // P

    def body(tt_ref, g_ref, pos_ref, type_ref, gam_ref, bet_ref, o_ref):
        tt = tt_ref[0, 0, :]
        typ = jnp.where((tt[:, None] == 0), type_ref[0, :][None, :],
                        type_ref[1, :][None, :])
        x = g_ref[...] + pos_ref[...] + typ
        mean = jnp.mean(x, axis=-1, keepdims=True)
        xc = x - mean
        var = jnp.mean(xc * xc, axis=-1, keepdims=True)
        inv = lax.rsqrt(var + EPS)
        o_ref[...] = xc * inv * gam_ref[0, :] + bet_ref[0, :]

    tok_blk = lambda i, j: (j * pos_blocks + i, 0)
    return pl.pallas_call(
        body,
        grid=(pos_blocks, batches),
        in_specs=[
            pl.BlockSpec((1, 1, tb), lambda i, j: (j * pos_blocks + i, 0, 0)),
            pl.BlockSpec((tb, D), tok_blk),
            pl.BlockSpec((tb, D), lambda i, j: (i, 0)),
            pl.BlockSpec((2, D), lambda i, j: (0, 0)),
            pl.BlockSpec((1, D), lambda i, j: (0, 0)),
            pl.BlockSpec((1, D), lambda i, j: (0, 0)),
        ],
        out_specs=pl.BlockSpec((tb, D), tok_blk),
        out_shape=jax.ShapeDtypeStruct((T, D), jnp.float32),
    )(tt3, gath, pos_emb, type_emb, gamma2, beta2)


def kernel(input_ids, token_type_ids, word_emb, pos_emb, type_emb, ln_gamma,
           ln_beta):
    B, S = input_ids.shape
    D = word_emb.shape[1]
    T = B * S

    n_ch, ch = 4, 64
    idx2d = input_ids.reshape(-1).astype(jnp.int32).reshape(-1, ch)
    gath = _sc_gather(word_emb, idx2d, n_ch, ch)

    tb = 256
    tt3 = token_type_ids.astype(jnp.int32).reshape(T // tb, 1, tb)
    out = _tc_finish(gath, tt3, pos_emb, type_emb,
                     ln_gamma.reshape(1, D), ln_beta.reshape(1, D), tb)
    return out.reshape(B, S, D)
